# baseline (speedup 1.0000x reference)
import functools

import jax
import jax.numpy as jnp
from jax import lax
from jax.experimental import pallas as pl
from jax.experimental.pallas import tpu as pltpu
from jax.experimental.pallas import tpu_sc as plsc

NUM_ENTITIES = 1000000
NUM_RELATIONS = 1000
D = 64
BATCH = 16384

NC, NS, L = 2, 16, 16
NW = NC * NS
CHUNK = 128
N_CHUNKS = BATCH // CHUNK
CPW = N_CHUNKS // NW


def _trig_body(rel_ref, cos_ref, sin_ref):
    th = rel_ref[...]
    cos_ref[...] = jnp.cos(th)
    sin_ref[...] = jnp.sin(th)


_trig = pl.pallas_call(
    _trig_body,
    out_shape=(
        jax.ShapeDtypeStruct((NUM_RELATIONS, D), jnp.float32),
        jax.ShapeDtypeStruct((NUM_RELATIONS, D), jnp.float32),
    ),
)


def _rotate_body(e1_ref, r_ref, ent_re, ent_im, cos_t, sin_t,
                 out_re, out_im, idx_e, idx_r, er, ei, cc, ss, sem):
    wid = lax.axis_index("s") * NC + lax.axis_index("c")
    row0 = wid * CPW
    pltpu.sync_copy(e1_ref.at[pl.ds(row0, CPW)], idx_e)
    pltpu.sync_copy(r_ref.at[pl.ds(row0, CPW)], idx_r)
    for j in range(CPW):
        cps = [
            pltpu.async_copy(ent_re.at[idx_e.at[j]], er, sem),
            pltpu.async_copy(ent_im.at[idx_e.at[j]], ei, sem),
            pltpu.async_copy(cos_t.at[idx_r.at[j]], cc, sem),
            pltpu.async_copy(sin_t.at[idx_r.at[j]], ss, sem),
        ]
        for c in cps:
            c.wait()

        def body(i, carry):
            for k in range(D // L):
                sl = pl.ds(k * L, L)
                a = er[i, sl]
                b = ei[i, sl]
                c = cc[i, sl]
                s = ss[i, sl]
                er[i, sl] = a * c - b * s
                ei[i, sl] = a * s + b * c
            return carry

        lax.fori_loop(0, CHUNK, body, 0)
        base = (row0 + j) * CHUNK
        pltpu.sync_copy(er, out_re.at[pl.ds(base, CHUNK)])
        pltpu.sync_copy(ei, out_im.at[pl.ds(base, CHUNK)])


_rotate = functools.partial(
    pl.kernel,
    out_type=(
        jax.ShapeDtypeStruct((BATCH, D), jnp.float32),
        jax.ShapeDtypeStruct((BATCH, D), jnp.float32),
    ),
    mesh=plsc.VectorSubcoreMesh(
        core_axis_name="c", subcore_axis_name="s", num_cores=NC, num_subcores=NS),
    scratch_types=[
        pltpu.VMEM((CPW, CHUNK), jnp.int32),
        pltpu.VMEM((CPW, CHUNK), jnp.int32),
        pltpu.VMEM((CHUNK, D), jnp.float32),
        pltpu.VMEM((CHUNK, D), jnp.float32),
        pltpu.VMEM((CHUNK, D), jnp.float32),
        pltpu.VMEM((CHUNK, D), jnp.float32),
        pltpu.SemaphoreType.DMA,
    ],
    compiler_params=pltpu.CompilerParams(use_tc_tiling_on_sc=False),
)(_rotate_body)


def kernel(e1, r, entity_real, entity_img, relation):
    e1 = e1.astype(jnp.int32).reshape(N_CHUNKS, CHUNK)
    r = r.astype(jnp.int32).reshape(N_CHUNKS, CHUNK)
    cos_t, sin_t = _trig(relation)
    out_re, out_im = _rotate(e1, r, entity_real, entity_img, cos_t, sin_t)
    return out_re, out_im
